# baseline (speedup 1.0000x reference)
import functools
import math

import jax
import jax.numpy as jnp
from jax import lax
from jax.experimental import pallas as pl
from jax.experimental.pallas import tpu as pltpu
from jax.experimental.pallas import tpu_sc as plsc

D_MODEL = 64
SCALE = math.sqrt(D_MODEL)

_info = plsc.get_sparse_core_info()
NC = _info.num_cores
NS = _info.num_subcores
LANES = _info.num_lanes
NW = NC * NS

V = 1000000
N_POS = 50
N_SEQ = 16384
N_IC = N_SEQ // 128
UIC = 2
UB = 128 * UIC
NBUF = 3

_mesh = plsc.VectorSubcoreMesh(core_axis_name="c", subcore_axis_name="s")


@functools.partial(
    pl.kernel,
    mesh=_mesh,
    compiler_params=pltpu.CompilerParams(
        use_tc_tiling_on_sc=False, needs_layout_passes=False),
    out_type=jax.ShapeDtypeStruct((N_POS, 8, N_IC, 8, 128), jnp.float32),
    scratch_types=[
        pltpu.VMEM((NBUF, UB), jnp.int32),
        pltpu.VMEM((NBUF, UB, D_MODEL), jnp.float32),
        pltpu.VMEM((NBUF, UIC, 8, 8, 129), jnp.float32),
        pltpu.SemaphoreType.DMA,
        pltpu.SemaphoreType.DMA,
        pltpu.SemaphoreType.DMA,
        pltpu.SemaphoreType.DMA,
        pltpu.SemaphoreType.DMA,
        pltpu.SemaphoreType.DMA,
        pltpu.SemaphoreType.DMA,
        pltpu.SemaphoreType.DMA,
        pltpu.SemaphoreType.DMA,
    ],
)
def _emb(x_t_hbm, table_hbm, out_hbm, xv, gbuf, outb,
         xs0, xs1, xs2, gs0, gs1, gs2, os0, os1, os2):
    wid = lax.axis_index("s") * NC + lax.axis_index("c")
    iota = lax.iota(jnp.int32, LANES)
    n_units = (N_POS * N_IC) // (UIC * NW)
    xsems = (xs0, xs1, xs2)
    gsems = (gs0, gs1, gs2)
    osems = (os0, os1, os2)

    def ji(t):
        u = t * NW + wid
        return u // (N_IC // UIC), (u % (N_IC // UIC)) * UIC

    def xv_copy(t, b):
        j, ic0 = ji(t)
        return pltpu.make_async_copy(
            x_t_hbm.at[j, pl.ds(ic0 * 128, UB)], xv.at[b], xsems[b])

    def gather(t, b):
        return pltpu.make_async_copy(
            table_hbm.at[xv.at[b]], gbuf.at[b], gsems[b])

    def out_copy(t, b, ich):
        j, ic0 = ji(t)
        return pltpu.make_async_copy(
            outb.at[b, ich, pl.ds(0, 8), pl.ds(0, 8), pl.ds(0, 128)],
            out_hbm.at[j, pl.ds(0, 8), ic0 + ich], osems[b])

    qhi = [lax.shift_right_logical(iota + qg * LANES, 3)
           for qg in range(D_MODEL // LANES)]
    qlo = [lax.bitwise_and(iota + qg * LANES,
                           jnp.full((LANES,), 7, jnp.int32))
           for qg in range(D_MODEL // LANES)]

    def transpose(b):
        @plsc.parallel_loop(0, UB, unroll=16)
        def l_body(l):
            ich = jnp.full((LANES,), l // 128, jnp.int32)
            col = jnp.full((LANES,), l % 128, jnp.int32)
            for qg in range(D_MODEL // LANES):
                vec = gbuf[b, l, pl.ds(qg * LANES, LANES)] * SCALE
                plsc.store_scatter(
                    outb.at[b], [ich, qhi[qg], qlo[qg], col], vec)

    xv_copy(0, 0).start()
    xv_copy(0, 0).wait()
    gather(0, 0).start()
    xv_copy(1, 1).start()

    def unit(t, carry):
        def body(b):
            nb = (b + 1) % NBUF
            nnb = (b + 2) % NBUF

            @pl.when(t + 2 < n_units)
            def _():
                xv_copy(t + 2, nnb).start()

            @pl.when(t + 1 < n_units)
            def _():
                xv_copy(t + 1, nb).wait()
                gather(t + 1, nb).start()

            @pl.when(t >= NBUF)
            def _():
                for ich in range(UIC):
                    out_copy(t - NBUF, b, ich).wait()

            gather(t, b).wait()
            transpose(b)
            for ich in range(UIC):
                out_copy(t, b, ich).start()

        b = lax.rem(t, NBUF)
        for bb in range(NBUF):
            @pl.when(b == bb)
            def _():
                body(bb)

        return carry

    lax.fori_loop(0, n_units, unit, 0)
    for t in range(n_units - NBUF, n_units):
        for ich in range(UIC):
            out_copy(t, t % NBUF, ich).wait()


def kernel(x, lut):
    out5 = _emb(x.T, lut)
    out_t = out5.transpose(0, 1, 3, 2, 4).reshape(N_POS, D_MODEL, N_SEQ)
    return out_t.transpose(2, 0, 1)


# speedup vs baseline: 1.0027x; 1.0027x over previous
import functools
import math

import jax
import jax.numpy as jnp
from jax import lax
from jax.experimental import pallas as pl
from jax.experimental.pallas import tpu as pltpu
from jax.experimental.pallas import tpu_sc as plsc

D_MODEL = 64
SCALE = math.sqrt(D_MODEL)

_info = plsc.get_sparse_core_info()
NC = _info.num_cores
NS = _info.num_subcores
LANES = _info.num_lanes
NW = NC * NS

V = 1000000
N_POS = 50
N_SEQ = 16384
N_IC = N_SEQ // 128
UIC = 2
UB = 128 * UIC
NBUF = 3

_mesh = plsc.VectorSubcoreMesh(core_axis_name="c", subcore_axis_name="s")


@functools.partial(
    pl.kernel,
    mesh=_mesh,
    compiler_params=pltpu.CompilerParams(
        use_tc_tiling_on_sc=False, needs_layout_passes=False),
    out_type=jax.ShapeDtypeStruct((N_POS, 8, N_IC, 8, 128), jnp.float32),
    scratch_types=[
        pltpu.VMEM((NBUF, UB), jnp.int32),
        pltpu.VMEM((NBUF, UB, D_MODEL), jnp.float32),
        pltpu.VMEM((NBUF, UIC, 8, 8, 129), jnp.float32),
        pltpu.SemaphoreType.DMA,
        pltpu.SemaphoreType.DMA,
        pltpu.SemaphoreType.DMA,
        pltpu.SemaphoreType.DMA,
        pltpu.SemaphoreType.DMA,
        pltpu.SemaphoreType.DMA,
        pltpu.SemaphoreType.DMA,
        pltpu.SemaphoreType.DMA,
        pltpu.SemaphoreType.DMA,
    ],
)
def _emb(x_t_hbm, table_hbm, out_hbm, xv, gbuf, outb,
         xs0, xs1, xs2, gs0, gs1, gs2, os0, os1, os2):
    wid = lax.axis_index("s") * NC + lax.axis_index("c")
    iota = lax.iota(jnp.int32, LANES)
    n_units = (N_POS * N_IC) // (UIC * NW)
    xsems = (xs0, xs1, xs2)
    gsems = (gs0, gs1, gs2)
    osems = (os0, os1, os2)

    def ji(t):
        u = t * NW + wid
        return u // (N_IC // UIC), (u % (N_IC // UIC)) * UIC

    def xv_copy(t, b):
        j, ic0 = ji(t)
        return pltpu.make_async_copy(
            x_t_hbm.at[j, pl.ds(ic0 * 128, UB)], xv.at[b], xsems[b])

    def gather(t, b):
        return pltpu.make_async_copy(
            table_hbm.at[xv.at[b]], gbuf.at[b], gsems[b])

    def out_copy(t, b, ich):
        j, ic0 = ji(t)
        return pltpu.make_async_copy(
            outb.at[b, ich, pl.ds(0, 8), pl.ds(0, 8), pl.ds(0, 128)],
            out_hbm.at[j, pl.ds(0, 8), ic0 + ich], osems[b])

    qhi = [lax.shift_right_logical(iota + qg * LANES, 3)
           for qg in range(D_MODEL // LANES)]
    qlo = [lax.bitwise_and(iota + qg * LANES,
                           jnp.full((LANES,), 7, jnp.int32))
           for qg in range(D_MODEL // LANES)]

    def transpose(b):
        @plsc.parallel_loop(0, UB, unroll=8)
        def l_body(l):
            ich = jnp.full((LANES,), l // 128, jnp.int32)
            col = jnp.full((LANES,), l % 128, jnp.int32)
            for qg in range(D_MODEL // LANES):
                vec = gbuf[b, l, pl.ds(qg * LANES, LANES)] * SCALE
                plsc.store_scatter(
                    outb.at[b], [ich, qhi[qg], qlo[qg], col], vec)

    xv_copy(0, 0).start()
    xv_copy(0, 0).wait()
    gather(0, 0).start()
    xv_copy(1, 1).start()

    def unit(t, carry):
        def body(b):
            nb = (b + 1) % NBUF
            nnb = (b + 2) % NBUF

            @pl.when(t + 2 < n_units)
            def _():
                xv_copy(t + 2, nnb).start()

            @pl.when(t + 1 < n_units)
            def _():
                xv_copy(t + 1, nb).wait()
                gather(t + 1, nb).start()

            @pl.when(t >= NBUF)
            def _():
                for ich in range(UIC):
                    out_copy(t - NBUF, b, ich).wait()

            gather(t, b).wait()
            transpose(b)
            for ich in range(UIC):
                out_copy(t, b, ich).start()

        b = lax.rem(t, NBUF)
        for bb in range(NBUF):
            @pl.when(b == bb)
            def _():
                body(bb)

        return carry

    lax.fori_loop(0, n_units, unit, 0)
    for t in range(n_units - NBUF, n_units):
        for ich in range(UIC):
            out_copy(t, t % NBUF, ich).wait()


def kernel(x, lut):
    out5 = _emb(x.T, lut)
    out_t = out5.transpose(0, 1, 3, 2, 4).reshape(N_POS, D_MODEL, N_SEQ)
    return out_t.transpose(2, 0, 1)
